# baseline (speedup 1.0000x reference)
import functools

import jax
import jax.numpy as jnp
from jax import lax
from jax.experimental import pallas as pl
from jax.experimental.pallas import tpu as pltpu
from jax.experimental.pallas import tpu_sc as plsc

_NUM_CODES = 8192
_DIM = 256
_NUM_TOKENS = 8192
_COMMITMENT_COST = 0.25


_BT1 = 256
_WIN = 4096


def _piece_bounds():
    edges = set(range(0, _NUM_CODES + 1, 512))
    edges.update(min(w, _NUM_CODES) for w in range(0, _NUM_CODES + _WIN, _WIN))
    edges = sorted(edges)
    return list(zip(edges[:-1], edges[1:]))


def _argmin_body(x_ref, w_ref, idx_ref):
    x = x_ref[...]
    xb = x.astype(jnp.bfloat16)
    x2 = jnp.sum(x * x, axis=1, keepdims=True)

    cur = jnp.full((_BT1,), jnp.inf, jnp.float32)
    idx = jnp.zeros((_BT1,), jnp.int32)
    wv = jnp.full((_BT1,), jnp.inf, jnp.float32)
    widx = jnp.full((_BT1,), _NUM_CODES, jnp.int32)
    for a, b in _piece_bounds():
        wp = w_ref[a:b, :]
        w2 = jnp.sum(wp * wp, axis=1)[None, :]
        xw = lax.dot_general(
            xb,
            wp.astype(jnp.bfloat16),
            (((1,), (1,)), ((), ())),
            preferred_element_type=jnp.float32,
        )
        dist = (x2 + w2) - 2.0 * xw
        pv = jnp.min(dist, axis=1)
        cols = lax.broadcasted_iota(jnp.int32, (_BT1, b - a), 1) + a
        pidx = jnp.min(jnp.where(dist == pv[:, None], cols, _NUM_CODES), axis=1)
        pbetter = pv < wv
        ptie = (pv == wv) & (pidx < widx)
        widx = jnp.where(pbetter | ptie, pidx, widx)
        wv = jnp.where(pbetter, pv, wv)
        if b % _WIN == 0 or b == _NUM_CODES:
            better = wv < cur
            tie = (wv == cur) & (widx < idx)
            idx = jnp.where(better | tie, widx, idx)
            cur = jnp.where(
                better, wv.astype(jnp.bfloat16).astype(jnp.float32), cur
            )
            wv = jnp.full((_BT1,), jnp.inf, jnp.float32)
            widx = jnp.full((_BT1,), _NUM_CODES, jnp.int32)
    idx_ref[0, 0, :] = idx


def _compute_indices(flat_x, weight):
    grid = (_NUM_TOKENS // _BT1,)
    out = pl.pallas_call(
        _argmin_body,
        grid=grid,
        in_specs=[
            pl.BlockSpec((_BT1, _DIM), lambda t: (t, 0)),
            pl.BlockSpec((_NUM_CODES, _DIM), lambda t: (0, 0)),
        ],
        out_specs=pl.BlockSpec((1, 1, _BT1), lambda t: (t, 0, 0)),
        out_shape=jax.ShapeDtypeStruct(
            (_NUM_TOKENS // _BT1, 1, _BT1), jnp.int32
        ),
    )(flat_x, weight)
    return out.reshape(_NUM_TOKENS)


_NW = 32
_B_PER_W = _NUM_TOKENS // _NW
_CHUNK = 128


def _gather_body(w_hbm, idx_hbm, out_hbm, idx_v, rows_v, sem):
    wid = lax.axis_index("s") * 2 + lax.axis_index("c")
    base = wid * _B_PER_W
    pltpu.sync_copy(idx_hbm.at[wid], idx_v)
    for j in range(_B_PER_W // _CHUNK):
        pltpu.async_copy(w_hbm.at[idx_v.at[j]], rows_v, sem).wait()
        pltpu.sync_copy(rows_v, out_hbm.at[pl.ds(base + j * _CHUNK, _CHUNK)])


def _gather_rows(weight, idx):
    mesh = plsc.VectorSubcoreMesh(core_axis_name="c", subcore_axis_name="s")
    k = pl.kernel(
        _gather_body,
        out_type=jax.ShapeDtypeStruct((_NUM_TOKENS, _DIM), jnp.float32),
        mesh=mesh,
        scratch_types=[
            pltpu.VMEM((_B_PER_W // _CHUNK, _CHUNK), jnp.int32),
            pltpu.VMEM((_CHUNK, _DIM), jnp.float32),
            pltpu.SemaphoreType.DMA,
        ],
    )
    return k(weight, idx.reshape(_NW, _B_PER_W // _CHUNK, _CHUNK))


_BT3 = 256
_BC3 = 2048


def _enc_body(
    idx_ref, q_ref, x_ref,
    enc_ref, qst_ref, loss_ref, perp_ref,
    counts_ref, acc_ref,
):
    t = pl.program_id(0)
    c = pl.program_id(1)
    nt = pl.num_programs(0)
    nc = pl.num_programs(1)

    idx = idx_ref[0, 0, :]
    codes = lax.broadcasted_iota(jnp.int32, (_BT3, _BC3), 1) + c * _BC3
    onehot = (idx[:, None] == codes).astype(jnp.float32)
    enc_ref[...] = onehot

    colsum = jnp.sum(onehot, axis=0, keepdims=True)

    @pl.when(t == 0)
    def _():
        counts_ref[:, pl.ds(c * _BC3, _BC3)] = colsum

    @pl.when(t > 0)
    def _():
        counts_ref[:, pl.ds(c * _BC3, _BC3)] += colsum

    @pl.when(c == 0)
    def _():
        q = q_ref[...]
        x = x_ref[...]
        qst_ref[...] = x + (q - x)
        part = jnp.sum((q - x) ** 2)

        @pl.when(t == 0)
        def _():
            acc_ref[0] = part

        @pl.when(t > 0)
        def _():
            acc_ref[0] += part

    @pl.when(t == nt - 1)
    def _():
        p = counts_ref[:, pl.ds(c * _BC3, _BC3)] * (1.0 / _NUM_TOKENS)
        ent = jnp.sum(p * jnp.log(p + 1e-10))

        @pl.when(c == 0)
        def _():
            acc_ref[1] = ent

        @pl.when(c > 0)
        def _():
            acc_ref[1] += ent

        @pl.when(c == nc - 1)
        def _():
            loss_ref[...] = jnp.full(
                (1, 1),
                _COMMITMENT_COST * (acc_ref[0] / (_NUM_TOKENS * _DIM)),
                jnp.float32,
            )
            perp_ref[...] = jnp.full((1, 1), jnp.exp(-acc_ref[1]), jnp.float32)


def _encodings_and_stats(idx3, quantized, flat_x):
    grid = (_NUM_TOKENS // _BT3, _NUM_CODES // _BC3)
    return pl.pallas_call(
        _enc_body,
        grid=grid,
        in_specs=[
            pl.BlockSpec((1, 1, _BT3), lambda t, c: (t, 0, 0)),
            pl.BlockSpec((_BT3, _DIM), lambda t, c: (t, 0)),
            pl.BlockSpec((_BT3, _DIM), lambda t, c: (t, 0)),
        ],
        out_specs=[
            pl.BlockSpec((_BT3, _BC3), lambda t, c: (t, c)),
            pl.BlockSpec((_BT3, _DIM), lambda t, c: (t, 0)),
            pl.BlockSpec((1, 1), lambda t, c: (0, 0)),
            pl.BlockSpec((1, 1), lambda t, c: (0, 0)),
        ],
        out_shape=[
            jax.ShapeDtypeStruct((_NUM_TOKENS, _NUM_CODES), jnp.float32),
            jax.ShapeDtypeStruct((_NUM_TOKENS, _DIM), jnp.float32),
            jax.ShapeDtypeStruct((1, 1), jnp.float32),
            jax.ShapeDtypeStruct((1, 1), jnp.float32),
        ],
        scratch_shapes=[
            pltpu.VMEM((1, _NUM_CODES), jnp.float32),
            pltpu.SMEM((2,), jnp.float32),
        ],
    )(idx3, quantized, flat_x)


def kernel(inputs, weight):
    input_shape = inputs.shape
    flat_x = inputs.reshape(_NUM_TOKENS, _DIM)
    idx = _compute_indices(flat_x, weight)
    quantized = _gather_rows(weight, idx)
    idx3 = idx.reshape(_NUM_TOKENS // _BT3, 1, _BT3)
    enc, qst, loss, perp = _encodings_and_stats(idx3, quantized, flat_x)
    return (
        loss.reshape(()),
        qst.reshape(input_shape),
        perp.reshape(()),
        enc,
    )


# speedup vs baseline: 1.0086x; 1.0086x over previous
import functools

import jax
import jax.numpy as jnp
from jax import lax
from jax.experimental import pallas as pl
from jax.experimental.pallas import tpu as pltpu
from jax.experimental.pallas import tpu_sc as plsc

_NUM_CODES = 8192
_DIM = 256
_NUM_TOKENS = 8192
_COMMITMENT_COST = 0.25


_BT1 = 256
_BC1 = 2048
_WIN = 4096
_PIECE = 512


def _enc_fused_body(x_ref, w_ref, idx_ref, enc_ref, perp_ref,
                    idx_s, counts_ref, acc_ref):
    t = pl.program_id(0)
    c = pl.program_id(1)
    nt = pl.num_programs(0)
    nc = pl.num_programs(1)

    @pl.when(c == 0)
    def _():
        x = x_ref[...]
        xb = x.astype(jnp.bfloat16)
        x2 = jnp.sum(x * x, axis=1, keepdims=True)

        cur = jnp.full((_BT1,), jnp.inf, jnp.float32)
        idx = jnp.zeros((_BT1,), jnp.int32)
        wv = jnp.full((_BT1,), jnp.inf, jnp.float32)
        widx = jnp.full((_BT1,), _NUM_CODES, jnp.int32)
        for a in range(0, _NUM_CODES, _PIECE):
            b = a + _PIECE
            wp = w_ref[a:b, :]
            w2 = jnp.sum(wp * wp, axis=1)[None, :]
            xw = lax.dot_general(
                xb,
                wp.astype(jnp.bfloat16),
                (((1,), (1,)), ((), ())),
                preferred_element_type=jnp.float32,
            )
            dist = (x2 + w2) - 2.0 * xw
            pv = jnp.min(dist, axis=1)
            cols = lax.broadcasted_iota(jnp.int32, (_BT1, _PIECE), 1) + a
            pidx = jnp.min(
                jnp.where(dist == pv[:, None], cols, _NUM_CODES), axis=1
            )
            pbetter = pv < wv
            ptie = (pv == wv) & (pidx < widx)
            widx = jnp.where(pbetter | ptie, pidx, widx)
            wv = jnp.where(pbetter, pv, wv)
            if b % _WIN == 0:
                better = wv < cur
                tie = (wv == cur) & (widx < idx)
                idx = jnp.where(better | tie, widx, idx)
                cur = jnp.where(
                    better, wv.astype(jnp.bfloat16).astype(jnp.float32), cur
                )
                wv = jnp.full((_BT1,), jnp.inf, jnp.float32)
                widx = jnp.full((_BT1,), _NUM_CODES, jnp.int32)
        idx_s[0, :] = idx
        idx_ref[0, 0, :] = idx

    idx = idx_s[0, :]
    codes = lax.broadcasted_iota(jnp.int32, (_BT1, _BC1), 1) + c * _BC1
    onehot = (idx[:, None] == codes).astype(jnp.float32)
    enc_ref[...] = onehot

    colsum = jnp.sum(onehot, axis=0, keepdims=True)

    @pl.when(t == 0)
    def _():
        counts_ref[:, pl.ds(c * _BC1, _BC1)] = colsum

    @pl.when(t > 0)
    def _():
        counts_ref[:, pl.ds(c * _BC1, _BC1)] += colsum

    @pl.when(t == nt - 1)
    def _():
        p = counts_ref[:, pl.ds(c * _BC1, _BC1)] * (1.0 / _NUM_TOKENS)
        ent = jnp.sum(p * jnp.log(p + 1e-10))

        @pl.when(c == 0)
        def _():
            acc_ref[0] = ent

        @pl.when(c > 0)
        def _():
            acc_ref[0] += ent

        @pl.when(c == nc - 1)
        def _():
            perp_ref[...] = jnp.full((1, 1), jnp.exp(-acc_ref[0]), jnp.float32)


def _indices_encodings_perplexity(flat_x, weight):
    grid = (_NUM_TOKENS // _BT1, _NUM_CODES // _BC1)
    idx3, enc, perp = pl.pallas_call(
        _enc_fused_body,
        grid=grid,
        in_specs=[
            pl.BlockSpec((_BT1, _DIM), lambda t, c: (t, 0)),
            pl.BlockSpec((_NUM_CODES, _DIM), lambda t, c: (0, 0)),
        ],
        out_specs=[
            pl.BlockSpec((1, 1, _BT1), lambda t, c: (t, 0, 0)),
            pl.BlockSpec((_BT1, _BC1), lambda t, c: (t, c)),
            pl.BlockSpec((1, 1), lambda t, c: (0, 0)),
        ],
        out_shape=[
            jax.ShapeDtypeStruct((_NUM_TOKENS // _BT1, 1, _BT1), jnp.int32),
            jax.ShapeDtypeStruct((_NUM_TOKENS, _NUM_CODES), jnp.float32),
            jax.ShapeDtypeStruct((1, 1), jnp.float32),
        ],
        scratch_shapes=[
            pltpu.VMEM((1, _BT1), jnp.int32),
            pltpu.VMEM((1, _NUM_CODES), jnp.float32),
            pltpu.SMEM((1,), jnp.float32),
        ],
    )(flat_x, weight)
    return idx3, enc, perp


_NW = 32
_B_PER_W = _NUM_TOKENS // _NW
_CHUNK = 128


def _gather_body(w_hbm, idx_hbm, out_hbm, idx_v, rows_v, sem):
    wid = lax.axis_index("s") * 2 + lax.axis_index("c")
    base = wid * _B_PER_W
    pltpu.sync_copy(idx_hbm.at[wid], idx_v)
    for j in range(_B_PER_W // _CHUNK):
        pltpu.async_copy(w_hbm.at[idx_v.at[j]], rows_v, sem).wait()
        pltpu.sync_copy(rows_v, out_hbm.at[pl.ds(base + j * _CHUNK, _CHUNK)])


def _gather_rows(weight, idx3):
    mesh = plsc.VectorSubcoreMesh(core_axis_name="c", subcore_axis_name="s")
    k = pl.kernel(
        _gather_body,
        out_type=jax.ShapeDtypeStruct((_NUM_TOKENS, _DIM), jnp.float32),
        mesh=mesh,
        scratch_types=[
            pltpu.VMEM((_B_PER_W // _CHUNK, _CHUNK), jnp.int32),
            pltpu.VMEM((_CHUNK, _DIM), jnp.float32),
            pltpu.SemaphoreType.DMA,
        ],
    )
    return k(weight, idx3.reshape(_NW, _B_PER_W // _CHUNK, _CHUNK))


_BT3 = 1024


def _loss_body(q_ref, x_ref, qst_ref, loss_ref, acc_ref):
    t = pl.program_id(0)
    nt = pl.num_programs(0)
    q = q_ref[...]
    x = x_ref[...]
    d = q - x
    qst_ref[...] = x + d
    part = jnp.sum(d * d)

    @pl.when(t == 0)
    def _():
        acc_ref[0] = part

    @pl.when(t > 0)
    def _():
        acc_ref[0] += part

    @pl.when(t == nt - 1)
    def _():
        loss_ref[...] = jnp.full(
            (1, 1),
            _COMMITMENT_COST * (acc_ref[0] / (_NUM_TOKENS * _DIM)),
            jnp.float32,
        )


def _st_and_loss(quantized, flat_x):
    grid = (_NUM_TOKENS // _BT3,)
    return pl.pallas_call(
        _loss_body,
        grid=grid,
        in_specs=[
            pl.BlockSpec((_BT3, _DIM), lambda t: (t, 0)),
            pl.BlockSpec((_BT3, _DIM), lambda t: (t, 0)),
        ],
        out_specs=[
            pl.BlockSpec((_BT3, _DIM), lambda t: (t, 0)),
            pl.BlockSpec((1, 1), lambda t: (0, 0)),
        ],
        out_shape=[
            jax.ShapeDtypeStruct((_NUM_TOKENS, _DIM), jnp.float32),
            jax.ShapeDtypeStruct((1, 1), jnp.float32),
        ],
        scratch_shapes=[pltpu.SMEM((1,), jnp.float32)],
    )(quantized, flat_x)


def kernel(inputs, weight):
    input_shape = inputs.shape
    flat_x = inputs.reshape(_NUM_TOKENS, _DIM)
    idx3, enc, perp = _indices_encodings_perplexity(flat_x, weight)
    quantized = _gather_rows(weight, idx3)
    qst, loss = _st_and_loss(quantized, flat_x)
    return (
        loss.reshape(()),
        qst.reshape(input_shape),
        perp.reshape(()),
        enc,
    )


# speedup vs baseline: 1.4999x; 1.4871x over previous
import functools

import jax
import jax.numpy as jnp
from jax import lax
from jax.experimental import pallas as pl
from jax.experimental.pallas import tpu as pltpu
from jax.experimental.pallas import tpu_sc as plsc

_NUM_CODES = 8192
_DIM = 256
_NUM_TOKENS = 8192
_COMMITMENT_COST = 0.25


_BT1 = 256
_BC1 = 2048
_WIN = 4096
_PIECE = 512


def _enc_fused_body(x_ref, w_ref, idx_ref, enc_ref, perp_ref,
                    idx_s, counts_ref, acc_ref, w2_ref, d_ref):
    t = pl.program_id(0)
    c = pl.program_id(1)
    nt = pl.num_programs(0)
    nc = pl.num_programs(1)

    @pl.when((t == 0) & (c == 0))
    def _():
        for a in range(0, _NUM_CODES, _PIECE):
            wp = w_ref[a:a + _PIECE, :]
            w2_ref[0, a:a + _PIECE] = jnp.sum(wp * wp, axis=1)

    @pl.when(c == 0)
    def _():
        x = x_ref[...]
        xb = x.astype(jnp.bfloat16)
        x2 = jnp.sum(x * x, axis=1, keepdims=True)

        cur = jnp.full((_BT1,), jnp.inf, jnp.float32)
        idx = jnp.zeros((_BT1,), jnp.int32)
        colsf_base = lax.broadcasted_iota(
            jnp.int32, (_BT1, _PIECE), 1
        ).astype(jnp.float32)
        for w0 in range(0, _NUM_CODES, _WIN):
            m = None
            for a in range(w0, w0 + _WIN, _PIECE):
                wp = w_ref[a:a + _PIECE, :]
                w2 = w2_ref[0, a:a + _PIECE][None, :]
                xw = lax.dot_general(
                    xb,
                    wp.astype(jnp.bfloat16),
                    (((1,), (1,)), ((), ())),
                    preferred_element_type=jnp.float32,
                )
                dist = (x2 + w2) - 2.0 * xw
                d_ref[:, a - w0:a - w0 + _PIECE] = dist
                m = dist if m is None else jnp.minimum(m, dist)
            wv = jnp.min(m, axis=1)
            ci = None
            for a in range(w0, w0 + _WIN, _PIECE):
                dist = d_ref[:, a - w0:a - w0 + _PIECE]
                colsf = colsf_base + float(a)
                cand = jnp.where(
                    dist == wv[:, None], colsf, float(_NUM_CODES)
                )
                ci = cand if ci is None else jnp.minimum(ci, cand)
            widx = jnp.min(ci, axis=1).astype(jnp.int32)
            better = wv < cur
            tie = (wv == cur) & (widx < idx)
            idx = jnp.where(better | tie, widx, idx)
            cur = jnp.where(
                better, wv.astype(jnp.bfloat16).astype(jnp.float32), cur
            )
        idx_s[0, :] = idx
        idx_ref[0, 0, :] = idx

    idx = idx_s[0, :]
    codes = lax.broadcasted_iota(jnp.int32, (_BT1, _BC1), 1) + c * _BC1
    onehot = (idx[:, None] == codes).astype(jnp.float32)
    enc_ref[...] = onehot

    colsum = jnp.sum(onehot, axis=0, keepdims=True)

    @pl.when(t == 0)
    def _():
        counts_ref[:, pl.ds(c * _BC1, _BC1)] = colsum

    @pl.when(t > 0)
    def _():
        counts_ref[:, pl.ds(c * _BC1, _BC1)] += colsum

    @pl.when(t == nt - 1)
    def _():
        p = counts_ref[:, pl.ds(c * _BC1, _BC1)] * (1.0 / _NUM_TOKENS)
        ent = jnp.sum(p * jnp.log(p + 1e-10))

        @pl.when(c == 0)
        def _():
            acc_ref[0] = ent

        @pl.when(c > 0)
        def _():
            acc_ref[0] += ent

        @pl.when(c == nc - 1)
        def _():
            perp_ref[...] = jnp.full((1, 1), jnp.exp(-acc_ref[0]), jnp.float32)


def _indices_encodings_perplexity(flat_x, weight):
    grid = (_NUM_TOKENS // _BT1, _NUM_CODES // _BC1)
    idx3, enc, perp = pl.pallas_call(
        _enc_fused_body,
        grid=grid,
        in_specs=[
            pl.BlockSpec((_BT1, _DIM), lambda t, c: (t, 0)),
            pl.BlockSpec((_NUM_CODES, _DIM), lambda t, c: (0, 0)),
        ],
        out_specs=[
            pl.BlockSpec((1, 1, _BT1), lambda t, c: (t, 0, 0)),
            pl.BlockSpec((_BT1, _BC1), lambda t, c: (t, c)),
            pl.BlockSpec((1, 1), lambda t, c: (0, 0)),
        ],
        out_shape=[
            jax.ShapeDtypeStruct((_NUM_TOKENS // _BT1, 1, _BT1), jnp.int32),
            jax.ShapeDtypeStruct((_NUM_TOKENS, _NUM_CODES), jnp.float32),
            jax.ShapeDtypeStruct((1, 1), jnp.float32),
        ],
        scratch_shapes=[
            pltpu.VMEM((1, _BT1), jnp.int32),
            pltpu.VMEM((1, _NUM_CODES), jnp.float32),
            pltpu.SMEM((1,), jnp.float32),
            pltpu.VMEM((1, _NUM_CODES), jnp.float32),
            pltpu.VMEM((_BT1, _WIN), jnp.float32),
        ],
    )(flat_x, weight)
    return idx3, enc, perp


_NW = 32
_B_PER_W = _NUM_TOKENS // _NW
_CHUNK = 128


def _gather_body(w_hbm, idx_hbm, out_hbm, idx_v, rows_v, sem):
    wid = lax.axis_index("s") * 2 + lax.axis_index("c")
    base = wid * _B_PER_W
    pltpu.sync_copy(idx_hbm.at[wid], idx_v)
    for j in range(_B_PER_W // _CHUNK):
        pltpu.async_copy(w_hbm.at[idx_v.at[j]], rows_v, sem).wait()
        pltpu.sync_copy(rows_v, out_hbm.at[pl.ds(base + j * _CHUNK, _CHUNK)])


def _gather_rows(weight, idx3):
    mesh = plsc.VectorSubcoreMesh(core_axis_name="c", subcore_axis_name="s")
    k = pl.kernel(
        _gather_body,
        out_type=jax.ShapeDtypeStruct((_NUM_TOKENS, _DIM), jnp.float32),
        mesh=mesh,
        scratch_types=[
            pltpu.VMEM((_B_PER_W // _CHUNK, _CHUNK), jnp.int32),
            pltpu.VMEM((_CHUNK, _DIM), jnp.float32),
            pltpu.SemaphoreType.DMA,
        ],
    )
    return k(weight, idx3.reshape(_NW, _B_PER_W // _CHUNK, _CHUNK))


_BT3 = 1024


def _loss_body(q_ref, x_ref, qst_ref, loss_ref, acc_ref):
    t = pl.program_id(0)
    nt = pl.num_programs(0)
    q = q_ref[...]
    x = x_ref[...]
    d = q - x
    qst_ref[...] = x + d
    part = jnp.sum(d * d)

    @pl.when(t == 0)
    def _():
        acc_ref[0] = part

    @pl.when(t > 0)
    def _():
        acc_ref[0] += part

    @pl.when(t == nt - 1)
    def _():
        loss_ref[...] = jnp.full(
            (1, 1),
            _COMMITMENT_COST * (acc_ref[0] / (_NUM_TOKENS * _DIM)),
            jnp.float32,
        )


def _st_and_loss(quantized, flat_x):
    grid = (_NUM_TOKENS // _BT3,)
    return pl.pallas_call(
        _loss_body,
        grid=grid,
        in_specs=[
            pl.BlockSpec((_BT3, _DIM), lambda t: (t, 0)),
            pl.BlockSpec((_BT3, _DIM), lambda t: (t, 0)),
        ],
        out_specs=[
            pl.BlockSpec((_BT3, _DIM), lambda t: (t, 0)),
            pl.BlockSpec((1, 1), lambda t: (0, 0)),
        ],
        out_shape=[
            jax.ShapeDtypeStruct((_NUM_TOKENS, _DIM), jnp.float32),
            jax.ShapeDtypeStruct((1, 1), jnp.float32),
        ],
        scratch_shapes=[pltpu.SMEM((1,), jnp.float32)],
    )(quantized, flat_x)


def kernel(inputs, weight):
    input_shape = inputs.shape
    flat_x = inputs.reshape(_NUM_TOKENS, _DIM)
    idx3, enc, perp = _indices_encodings_perplexity(flat_x, weight)
    quantized = _gather_rows(weight, idx3)
    qst, loss = _st_and_loss(quantized, flat_x)
    return (
        loss.reshape(()),
        qst.reshape(input_shape),
        perp.reshape(()),
        enc,
    )


# speedup vs baseline: 1.8131x; 1.2088x over previous
import functools

import jax
import jax.numpy as jnp
from jax import lax
from jax.experimental import pallas as pl
from jax.experimental.pallas import tpu as pltpu
from jax.experimental.pallas import tpu_sc as plsc

_NUM_CODES = 8192
_DIM = 256
_NUM_TOKENS = 8192
_COMMITMENT_COST = 0.25


_BT1 = 512
_BC1 = 2048
_WIN = 4096
_PIECE = 512


def _enc_fused_body(x_ref, w_ref, idx_ref, enc_ref, perp_ref,
                    idx_s, counts_ref, acc_ref, w2_ref, d_ref):
    t = pl.program_id(0)
    c = pl.program_id(1)
    nt = pl.num_programs(0)
    nc = pl.num_programs(1)

    @pl.when((t == 0) & (c == 0))
    def _():
        for a in range(0, _NUM_CODES, _PIECE):
            wp = w_ref[a:a + _PIECE, :]
            w2_ref[0, a:a + _PIECE] = jnp.sum(wp * wp, axis=1)

    @pl.when(c == 0)
    def _():
        x = x_ref[...]
        xb = x.astype(jnp.bfloat16)
        x2 = jnp.sum(x * x, axis=1, keepdims=True)

        cur = jnp.full((_BT1,), jnp.inf, jnp.float32)
        idx = jnp.zeros((_BT1,), jnp.int32)
        colsf_base = lax.broadcasted_iota(
            jnp.int32, (_BT1, _PIECE), 1
        ).astype(jnp.float32)
        for w0 in range(0, _NUM_CODES, _WIN):
            m = None
            for a in range(w0, w0 + _WIN, _PIECE):
                wp = w_ref[a:a + _PIECE, :]
                w2 = w2_ref[0, a:a + _PIECE][None, :]
                xw = lax.dot_general(
                    xb,
                    wp.astype(jnp.bfloat16),
                    (((1,), (1,)), ((), ())),
                    preferred_element_type=jnp.float32,
                )
                dist = (x2 + w2) - 2.0 * xw
                d_ref[:, a - w0:a - w0 + _PIECE] = dist
                m = dist if m is None else jnp.minimum(m, dist)
            wv = jnp.min(m, axis=1)
            ci = None
            for a in range(w0, w0 + _WIN, _PIECE):
                dist = d_ref[:, a - w0:a - w0 + _PIECE]
                colsf = colsf_base + float(a)
                cand = jnp.where(
                    dist == wv[:, None], colsf, float(_NUM_CODES)
                )
                ci = cand if ci is None else jnp.minimum(ci, cand)
            widx = jnp.min(ci, axis=1).astype(jnp.int32)
            better = wv < cur
            tie = (wv == cur) & (widx < idx)
            idx = jnp.where(better | tie, widx, idx)
            cur = jnp.where(
                better, wv.astype(jnp.bfloat16).astype(jnp.float32), cur
            )
        idx_s[0, :] = idx
        idx_ref[0, 0, :] = idx

    idx = idx_s[0, :]
    codes = lax.broadcasted_iota(jnp.int32, (_BT1, _BC1), 1) + c * _BC1
    onehot = (idx[:, None] == codes).astype(jnp.float32)
    enc_ref[...] = onehot

    colsum = jnp.sum(onehot, axis=0, keepdims=True)

    @pl.when(t == 0)
    def _():
        counts_ref[:, pl.ds(c * _BC1, _BC1)] = colsum

    @pl.when(t > 0)
    def _():
        counts_ref[:, pl.ds(c * _BC1, _BC1)] += colsum

    @pl.when(t == nt - 1)
    def _():
        p = counts_ref[:, pl.ds(c * _BC1, _BC1)] * (1.0 / _NUM_TOKENS)
        ent = jnp.sum(p * jnp.log(p + 1e-10))

        @pl.when(c == 0)
        def _():
            acc_ref[0] = ent

        @pl.when(c > 0)
        def _():
            acc_ref[0] += ent

        @pl.when(c == nc - 1)
        def _():
            perp_ref[...] = jnp.full((1, 1), jnp.exp(-acc_ref[0]), jnp.float32)


def _indices_encodings_perplexity(flat_x, weight):
    grid = (_NUM_TOKENS // _BT1, _NUM_CODES // _BC1)
    idx3, enc, perp = pl.pallas_call(
        _enc_fused_body,
        grid=grid,
        in_specs=[
            pl.BlockSpec((_BT1, _DIM), lambda t, c: (t, 0)),
            pl.BlockSpec((_NUM_CODES, _DIM), lambda t, c: (0, 0)),
        ],
        out_specs=[
            pl.BlockSpec((1, 1, _BT1), lambda t, c: (t, 0, 0)),
            pl.BlockSpec((_BT1, _BC1), lambda t, c: (t, c)),
            pl.BlockSpec((1, 1), lambda t, c: (0, 0)),
        ],
        out_shape=[
            jax.ShapeDtypeStruct((_NUM_TOKENS // _BT1, 1, _BT1), jnp.int32),
            jax.ShapeDtypeStruct((_NUM_TOKENS, _NUM_CODES), jnp.float32),
            jax.ShapeDtypeStruct((1, 1), jnp.float32),
        ],
        scratch_shapes=[
            pltpu.VMEM((1, _BT1), jnp.int32),
            pltpu.VMEM((1, _NUM_CODES), jnp.float32),
            pltpu.SMEM((1,), jnp.float32),
            pltpu.VMEM((1, _NUM_CODES), jnp.float32),
            pltpu.VMEM((_BT1, _WIN), jnp.float32),
        ],
    )(flat_x, weight)
    return idx3, enc, perp


_NW = 32
_B_PER_W = _NUM_TOKENS // _NW
_CHUNK = 128


def _gather_body(w_hbm, idx_hbm, out_hbm, idx_v, rows_v, sem):
    wid = lax.axis_index("s") * 2 + lax.axis_index("c")
    base = wid * _B_PER_W
    pltpu.sync_copy(idx_hbm.at[wid], idx_v)
    for j in range(_B_PER_W // _CHUNK):
        pltpu.async_copy(w_hbm.at[idx_v.at[j]], rows_v, sem).wait()
        pltpu.sync_copy(rows_v, out_hbm.at[pl.ds(base + j * _CHUNK, _CHUNK)])


def _gather_rows(weight, idx3):
    mesh = plsc.VectorSubcoreMesh(core_axis_name="c", subcore_axis_name="s")
    k = pl.kernel(
        _gather_body,
        out_type=jax.ShapeDtypeStruct((_NUM_TOKENS, _DIM), jnp.float32),
        mesh=mesh,
        scratch_types=[
            pltpu.VMEM((_B_PER_W // _CHUNK, _CHUNK), jnp.int32),
            pltpu.VMEM((_CHUNK, _DIM), jnp.float32),
            pltpu.SemaphoreType.DMA,
        ],
    )
    return k(weight, idx3.reshape(_NW, _B_PER_W // _CHUNK, _CHUNK))


_BT3 = 1024


def _loss_body(q_ref, x_ref, qst_ref, loss_ref, acc_ref):
    t = pl.program_id(0)
    nt = pl.num_programs(0)
    q = q_ref[...]
    x = x_ref[...]
    d = q - x
    qst_ref[...] = x + d
    part = jnp.sum(d * d)

    @pl.when(t == 0)
    def _():
        acc_ref[0] = part

    @pl.when(t > 0)
    def _():
        acc_ref[0] += part

    @pl.when(t == nt - 1)
    def _():
        loss_ref[...] = jnp.full(
            (1, 1),
            _COMMITMENT_COST * (acc_ref[0] / (_NUM_TOKENS * _DIM)),
            jnp.float32,
        )


def _st_and_loss(quantized, flat_x):
    grid = (_NUM_TOKENS // _BT3,)
    return pl.pallas_call(
        _loss_body,
        grid=grid,
        in_specs=[
            pl.BlockSpec((_BT3, _DIM), lambda t: (t, 0)),
            pl.BlockSpec((_BT3, _DIM), lambda t: (t, 0)),
        ],
        out_specs=[
            pl.BlockSpec((_BT3, _DIM), lambda t: (t, 0)),
            pl.BlockSpec((1, 1), lambda t: (0, 0)),
        ],
        out_shape=[
            jax.ShapeDtypeStruct((_NUM_TOKENS, _DIM), jnp.float32),
            jax.ShapeDtypeStruct((1, 1), jnp.float32),
        ],
        scratch_shapes=[pltpu.SMEM((1,), jnp.float32)],
    )(quantized, flat_x)


def kernel(inputs, weight):
    input_shape = inputs.shape
    flat_x = inputs.reshape(_NUM_TOKENS, _DIM)
    idx3, enc, perp = _indices_encodings_perplexity(flat_x, weight)
    quantized = _gather_rows(weight, idx3)
    qst, loss = _st_and_loss(quantized, flat_x)
    return (
        loss.reshape(()),
        qst.reshape(input_shape),
        perp.reshape(()),
        enc,
    )
